# baseline (speedup 1.0000x reference)
import functools

import jax
import jax.numpy as jnp
from jax import lax
from jax.experimental import pallas as pl
from jax.experimental.pallas import tpu as pltpu
from jax.experimental.pallas import tpu_sc as plsc

_HOP = 512
_L = 16
_NBUF = 3


def _sc_body(x_hbm, a_hbm, s_hbm, o_hbm, b0, b1, b2, wall, abuf, sbuf,
             *sems, hop, per_worker, chunk, num_cores):
    wid = lax.axis_index("s") * num_cores + lax.axis_index("c")
    even = (wid % 2) == 0
    bufs = (b0, b1, b2)
    ld_sems, st_sems = sems[:_NBUF], sems[_NBUF:]
    nchunks = per_worker // chunk
    ppc = chunk // hop
    kregs = hop // _L

    pltpu.sync_copy(a_hbm, abuf)
    pltpu.sync_copy(s_hbm, sbuf)
    for k in range(kregs):
        i = k * _L
        wlo = abuf[pl.ds(i, _L)] * sbuf[pl.ds(i, _L)]
        whi = abuf[pl.ds(hop + i, _L)] * sbuf[pl.ds(hop + i, _L)]
        wall[pl.ds(i, _L)] = wlo
        wall[pl.ds(hop + i, _L)] = wlo + whi
        wall[pl.ds(2 * hop + i, _L)] = whi

    def base(c):
        return wid * per_worker + c * chunk

    def compute(buf, c):
        p_lo = 1 if c == 0 else 0
        p_hi = ppc - 1 if c == nchunks - 1 else ppc

        @plsc.parallel_loop(p_lo, p_hi, unroll=2)
        def _(p):
            row = p * hop
            for k in range(kregs):
                i = row + k * _L
                buf[pl.ds(i, _L)] = (
                    buf[pl.ds(i, _L)] * wall[pl.ds(hop + k * _L, _L)])

        if c == 0:
            off = jnp.where(even, 0, hop)
            for k in range(kregs):
                i = k * _L
                buf[pl.ds(i, _L)] = (
                    buf[pl.ds(i, _L)] * wall[pl.ds(off + k * _L, _L)])
        if c == nchunks - 1:
            off = jnp.where(even, hop, 2 * hop)
            row = (ppc - 1) * hop
            for k in range(kregs):
                i = row + k * _L
                buf[pl.ds(i, _L)] = (
                    buf[pl.ds(i, _L)] * wall[pl.ds(off + k * _L, _L)])

    ld = [None] * nchunks
    st = [None] * nchunks
    for c in range(min(_NBUF, nchunks)):
        ld[c] = pltpu.async_copy(
            x_hbm.at[pl.ds(base(c), chunk)], bufs[c % _NBUF],
            ld_sems[c % _NBUF])
    for c in range(nchunks):
        buf = bufs[c % _NBUF]
        ld[c].wait()
        compute(buf, c)
        st[c] = pltpu.async_copy(
            buf, o_hbm.at[pl.ds(base(c), chunk)], st_sems[c % _NBUF])
        nxt = c + _NBUF
        if nxt < nchunks:
            st[nxt - _NBUF].wait()
            ld[nxt] = pltpu.async_copy(
                x_hbm.at[pl.ds(base(nxt), chunk)], bufs[nxt % _NBUF],
                ld_sems[nxt % _NBUF])
    for c in range(max(0, nchunks - _NBUF), nchunks):
        st[c].wait()


def kernel(x, analysis_window, synthesis_window):
    batch, num_samples = x.shape
    seg = analysis_window.shape[-1]
    hop = _HOP
    assert seg == 2 * hop and num_samples % hop == 0
    num_cores, num_subcores = 2, 16
    nw = num_cores * num_subcores
    total = batch * num_samples
    per_worker = total // nw
    assert num_samples % per_worker == 0
    chunk = 32768

    body = functools.partial(
        _sc_body, hop=hop, per_worker=per_worker, chunk=chunk,
        num_cores=num_cores)
    out = pl.kernel(
        body,
        mesh=plsc.VectorSubcoreMesh(
            core_axis_name="c", subcore_axis_name="s", num_cores=num_cores),
        out_type=jax.ShapeDtypeStruct((total,), x.dtype),
        scratch_types=[
            pltpu.VMEM((chunk,), jnp.float32),
            pltpu.VMEM((chunk,), jnp.float32),
            pltpu.VMEM((chunk,), jnp.float32),
            pltpu.VMEM((3 * hop,), jnp.float32),
            pltpu.VMEM((seg,), jnp.float32),
            pltpu.VMEM((seg,), jnp.float32),
        ] + [pltpu.SemaphoreType.DMA] * (2 * _NBUF),
    )(x.reshape(total), analysis_window, synthesis_window)
    return out.reshape(batch, num_samples)
